# baseline (speedup 1.0000x reference)
import functools

import jax
import jax.numpy as jnp
from jax import lax
from jax.experimental import pallas as pl
from jax.experimental.pallas import tpu as pltpu
from jax.experimental.pallas import tpu_sc as plsc

B = 1024
D = 2048
C = 1000
CPAD = 1024
K = 16
NUM_DROP = 512
IDX_PAD = 128

SC_CORES = 2
SC_SUBCORES = 16
NW = SC_CORES * SC_SUBCORES
RPW = B // NW


def _topk_idx_body(p_ref, y_ref, out_ref):
    p = p_ref[...]
    rb = p.shape[0]
    col = lax.broadcasted_iota(jnp.int32, (rb, CPAD), 1)
    ocol = lax.broadcasted_iota(jnp.int32, (rb, IDX_PAD), 1)
    acc = jnp.broadcast_to(y_ref[...], (rb, IDX_PAD))
    for k in range(K):
        m = jnp.max(p, axis=1, keepdims=True)
        idx = jnp.min(jnp.where(p == m, col, CPAD), axis=1, keepdims=True)
        acc = jnp.where(ocol == k, idx, acc)
        p = jnp.where(col == idx, -jnp.inf, p)
    out_ref[...] = acc


def _topk_indices(p_pad, y2):
    rb = 128
    return pl.pallas_call(
        _topk_idx_body,
        grid=(B // rb,),
        in_specs=[
            pl.BlockSpec((rb, CPAD), lambda i: (i, 0)),
            pl.BlockSpec((rb, 1), lambda i: (i, 0)),
        ],
        out_specs=pl.BlockSpec((rb, IDX_PAD), lambda i: (i, 0)),
        out_shape=jax.ShapeDtypeStruct((B, IDX_PAD), jnp.int32),
    )(p_pad, y2)


CHUNK = 8


NVEC = D // 16


def _count_ge(src, n, bitmask, iota16):
    nvec = (n + 15) >> 4

    def body(j, acc):
        off = j * 16
        v = src[pl.ds(off, 16)]
        valid = iota16 < (n - off)
        m = jnp.logical_and((v & bitmask) != 0, valid)
        return acc + jnp.where(m, 1, 0)

    acc = lax.fori_loop(0, nvec, body, jnp.zeros((16,), jnp.int32))
    return jnp.sum(acc)


def _compact(src, dst, n, bitmask, choose_set, iota16):
    nvec = (n + 15) >> 4

    def body(j, basev):
        off = j * 16
        v = src[pl.ds(off, 16)]
        valid = iota16 < (n - off)
        bit_on = (v & bitmask) != 0
        want = jnp.logical_and(bit_on == choose_set, valid)
        ones = jnp.where(want, 1, 0)
        inc = plsc.cumsum(ones)
        pos = basev + inc - 1
        plsc.store_scatter(dst, [pos], v, mask=want)
        return basev + plsc.all_reduce_population_count(want)

    lax.fori_loop(0, nvec, body, jnp.zeros((16,), jnp.int32))


def _sc_fused_body(idx_hbm, y_hbm, w_hbm, x_hbm, out_hbm,
                   idxbuf, yidx, ybuf, wbuf0, wbuf1,
                   srow, canda, candb, xrow0, xrow1, orow0, orow1,
                   sem_i, sem_y, sem_w0, sem_w1,
                   sem_x0, sem_x1, sem_o0, sem_o1):
    wid = lax.axis_index("s") * SC_CORES + lax.axis_index("c")
    base = wid * RPW
    wbufs = (wbuf0, wbuf1)
    sem_ws = (sem_w0, sem_w1)
    xrows = (xrow0, xrow1)
    sem_xs = (sem_x0, sem_x1)
    orows = (orow0, orow1)
    sem_os = (sem_o0, sem_o1)
    nchunks = RPW // CHUNK
    iota16 = lax.iota(jnp.int32, 16)

    pltpu.async_copy(idx_hbm.at[pl.ds(base, RPW)], idxbuf, sem_i).wait()
    pltpu.async_copy(y_hbm.at[pl.ds(base, RPW)], yidx, sem_i).wait()

    def gather_w(r, slot):
        pltpu.async_copy(
            w_hbm.at[idxbuf.at[r, pl.ds(0, K)]], wbufs[slot], sem_ws[slot])

    def fetch_x(r, slot):
        pltpu.async_copy(x_hbm.at[base + r], xrows[slot], sem_xs[slot])

    gather_w(0, 0)
    fetch_x(0, 0)
    fetch_x(1, 1)

    def chunk_body(c, carry_c):
        pltpu.async_copy(
            w_hbm.at[yidx.at[pl.ds(c * CHUNK, CHUNK)]], ybuf, sem_y).wait()

        for i in range(CHUNK):
            r = c * CHUNK + i
            slot = i % 2
            wbuf = wbufs[slot]
            pltpu.make_async_copy(
                w_hbm.at[idxbuf.at[r, pl.ds(0, K)]], wbuf, sem_ws[slot]).wait()
            if i < CHUNK - 1:
                gather_w(r + 1, 1 - slot)
            else:
                @pl.when(c < nchunks - 1)
                def _():
                    gather_w(r + 1, 1 - slot)

            def col_body(j, mm, wbuf=wbuf, ybuf=ybuf, i=i):
                sl = pl.ds(j * 16, 16)
                hi = wbuf[0, sl]
                lo = hi
                for k in range(1, K):
                    v = wbuf[k, sl]
                    hi = jnp.maximum(hi, v)
                    lo = jnp.minimum(lo, v)
                wy = ybuf[i, sl]
                s = jnp.maximum(hi - wy, wy - lo)
                srow[sl] = s
                canda[sl] = lax.bitcast_convert_type(s, jnp.int32)
                gmax, gmin = mm
                return (jnp.maximum(gmax, s), jnp.minimum(gmin, s))

            gmax, gmin = lax.fori_loop(
                0, NVEC, col_body,
                (jnp.zeros((16,), jnp.float32),
                 jnp.full((16,), jnp.inf, jnp.float32)))

            maxbits = lax.bitcast_convert_type(jnp.max(gmax), jnp.int32)
            minbits = lax.bitcast_convert_type(jnp.min(gmin), jnp.int32)
            diff = maxbits ^ minbits
            eb = (lax.bitcast_convert_type(diff.astype(jnp.float32),
                                           jnp.int32) >> 23) & 0xFF
            b0 = jnp.clip(eb - 127, 0, 30)
            low = (1 << (b0 + 1)) - 1
            prefix0 = jnp.where(diff == 0, minbits, minbits & ~low)
            rounds = jnp.where(diff == 0, 0, b0 + 1)

            def round_pair(t, carry):
                prefix, n_gt, n = carry
                bit = b0 - 2 * t
                bm = 1 << bit
                n1 = _count_ge(canda, n, bm, iota16)
                ge = n_gt + n1 >= NUM_DROP
                _compact(canda, candb, n, bm, ge, iota16)
                prefix = jnp.where(ge, prefix | bm, prefix)
                n_gt = jnp.where(ge, n_gt, n_gt + n1)
                n = jnp.where(ge, n1, n - n1)
                act = 2 * t + 1 < rounds
                bit2 = jnp.maximum(b0 - (2 * t + 1), 0)
                bm2 = 1 << bit2
                n2 = _count_ge(candb, n, bm2, iota16)
                ge2 = n_gt + n2 >= NUM_DROP
                _compact(candb, canda, n, bm2, ge2, iota16)
                prefix = jnp.where(jnp.logical_and(act, ge2),
                                   prefix | bm2, prefix)
                n_gt = jnp.where(jnp.logical_and(act, jnp.logical_not(ge2)),
                                 n_gt + n2, n_gt)
                n = jnp.where(act, jnp.where(ge2, n2, n - n2), n)
                return (prefix, n_gt, n)

            prefix, _, _ = lax.fori_loop(
                0, (rounds + 1) >> 1, round_pair,
                (prefix0, jnp.int32(0), jnp.int32(D)))

            xrow = xrows[slot]
            orow = orows[slot]
            pltpu.make_async_copy(
                x_hbm.at[base + r], xrow, sem_xs[slot]).wait()
            if i >= 2:
                pltpu.make_async_copy(
                    orow, out_hbm.at[base + r - 2], sem_os[slot]).wait()
            else:
                @pl.when(c > 0)
                def _():
                    pltpu.make_async_copy(
                        orow, out_hbm.at[base + r - 2], sem_os[slot]).wait()

            pv = jnp.full((16,), prefix, jnp.int32)

            def mask_body(j, carry2, xrow=xrow, orow=orow, pv=pv):
                sl = pl.ds(j * 16, 16)
                sb = lax.bitcast_convert_type(srow[sl], jnp.int32)
                orow[sl] = jnp.where(sb >= pv, 0.0, xrow[sl])
                return carry2

            lax.fori_loop(0, NVEC, mask_body, 0)
            pltpu.async_copy(orow, out_hbm.at[base + r], sem_os[slot])
            if i + 2 < CHUNK:
                fetch_x(r + 2, slot)
            else:
                @pl.when(c < nchunks - 1)
                def _():
                    fetch_x(r + 2, slot)
        return carry_c

    lax.fori_loop(0, nchunks, chunk_body, 0)

    for i in range(2):
        pltpu.make_async_copy(
            orows[i], out_hbm.at[base + RPW - 2 + i], sem_os[i]).wait()


@functools.cache
def _sc_fused_kernel():
    mesh = plsc.VectorSubcoreMesh(
        core_axis_name="c", subcore_axis_name="s",
        num_cores=SC_CORES, num_subcores=SC_SUBCORES)
    return pl.kernel(
        _sc_fused_body,
        out_type=jax.ShapeDtypeStruct((B, D), jnp.float32),
        mesh=mesh,
        compiler_params=pltpu.CompilerParams(needs_layout_passes=False),
        scratch_types=[
            pltpu.VMEM((RPW, IDX_PAD), jnp.int32),
            pltpu.VMEM((RPW,), jnp.int32),
            pltpu.VMEM((CHUNK, D), jnp.float32),
            pltpu.VMEM((K, D), jnp.float32),
            pltpu.VMEM((K, D), jnp.float32),
            pltpu.VMEM((D,), jnp.float32),
            pltpu.VMEM((D,), jnp.int32),
            pltpu.VMEM((D,), jnp.int32),
            pltpu.VMEM((D,), jnp.float32),
            pltpu.VMEM((D,), jnp.float32),
            pltpu.VMEM((D,), jnp.float32),
            pltpu.VMEM((D,), jnp.float32),
            pltpu.SemaphoreType.DMA,
            pltpu.SemaphoreType.DMA,
            pltpu.SemaphoreType.DMA,
            pltpu.SemaphoreType.DMA,
            pltpu.SemaphoreType.DMA,
            pltpu.SemaphoreType.DMA,
            pltpu.SemaphoreType.DMA,
            pltpu.SemaphoreType.DMA,
        ],
    )


def kernel(x, y, weight_matrix, prev_output):
    y1 = y.astype(jnp.int32)
    p_pad = jnp.pad(prev_output, ((0, 0), (0, CPAD - C)),
                    constant_values=-jnp.inf)
    idx = _topk_indices(p_pad, y1.reshape(B, 1))
    return _sc_fused_kernel()(idx, y1, weight_matrix, x)


# speedup vs baseline: 2.4754x; 2.4754x over previous
import functools

import jax
import jax.numpy as jnp
from jax import lax
from jax.experimental import pallas as pl
from jax.experimental.pallas import tpu as pltpu
from jax.experimental.pallas import tpu_sc as plsc

B = 1024
D = 2048
C = 1000
CPAD = 1024
K = 16
NUM_DROP = 512
IDX_PAD = 128

SC_CORES = 2
SC_SUBCORES = 16
NW = SC_CORES * SC_SUBCORES
RPW = B // NW
CHUNK = 8


def _topk_idx_body(p_ref, y_ref, out_ref):
    p = p_ref[...]
    rb = p.shape[0]
    col = lax.broadcasted_iota(jnp.int32, (rb, CPAD), 1)
    ocol = lax.broadcasted_iota(jnp.int32, (rb, IDX_PAD), 1)
    acc = jnp.broadcast_to(y_ref[...], (rb, IDX_PAD))
    for k in range(K):
        m = jnp.max(p, axis=1, keepdims=True)
        idx = jnp.min(jnp.where(p == m, col, CPAD), axis=1, keepdims=True)
        acc = jnp.where(ocol == k, idx, acc)
        p = jnp.where(col == idx, -jnp.inf, p)
    out_ref[...] = acc


def _topk_indices(p_pad, y2):
    rb = 128
    return pl.pallas_call(
        _topk_idx_body,
        grid=(B // rb,),
        in_specs=[
            pl.BlockSpec((rb, CPAD), lambda i: (i, 0)),
            pl.BlockSpec((rb, 1), lambda i: (i, 0)),
        ],
        out_specs=pl.BlockSpec((rb, IDX_PAD), lambda i: (i, 0)),
        out_shape=jax.ShapeDtypeStruct((B, IDX_PAD), jnp.int32),
    )(p_pad, y2)


def _sc_scores_body(idx_hbm, y_hbm, w_hbm, scores_hbm,
                    idxbuf, yidx, ybuf, wbuf0, wbuf1, srow0, srow1,
                    sem_i, sem_y, sem_w0, sem_w1, sem_s0, sem_s1):
    wid = lax.axis_index("s") * SC_CORES + lax.axis_index("c")
    base = wid * RPW
    wbufs = (wbuf0, wbuf1)
    sem_ws = (sem_w0, sem_w1)
    srows = (srow0, srow1)
    sem_ss = (sem_s0, sem_s1)

    pltpu.async_copy(idx_hbm.at[pl.ds(base, RPW)], idxbuf, sem_i).wait()
    pltpu.async_copy(y_hbm.at[pl.ds(base, RPW)], yidx, sem_i).wait()

    def gather_w(r, slot):
        pltpu.async_copy(
            w_hbm.at[idxbuf.at[r, pl.ds(0, K)]], wbufs[slot], sem_ws[slot])

    gather_w(0, 0)

    def chunk_body(c, carry):
        pltpu.async_copy(
            w_hbm.at[yidx.at[pl.ds(c * CHUNK, CHUNK)]], ybuf, sem_y).wait()

        for i in range(CHUNK):
            r = c * CHUNK + i
            slot = i % 2
            wbuf = wbufs[slot]
            pltpu.make_async_copy(
                w_hbm.at[idxbuf.at[r, pl.ds(0, K)]], wbuf, sem_ws[slot]).wait()
            if i < CHUNK - 1:
                gather_w(r + 1, 1 - slot)
            else:
                @pl.when(c < RPW // CHUNK - 1)
                def _():
                    gather_w(r + 1, 1 - slot)

            srow = srows[slot]
            if i >= 2:
                pltpu.make_async_copy(
                    srow, scores_hbm.at[base + r - 2], sem_ss[slot]).wait()
            else:
                @pl.when(c > 0)
                def _():
                    pltpu.make_async_copy(
                        srow, scores_hbm.at[base + r - 2], sem_ss[slot]).wait()

            def col_body(j, carry2, wbuf=wbuf, srow=srow, i=i):
                sl = pl.ds(j * 16, 16)
                hi = wbuf[0, sl]
                lo = hi
                for k in range(1, K):
                    v = wbuf[k, sl]
                    hi = jnp.maximum(hi, v)
                    lo = jnp.minimum(lo, v)
                wy = ybuf[i, sl]
                srow[sl] = jnp.maximum(hi - wy, wy - lo)
                return carry2

            lax.fori_loop(0, D // 16, col_body, 0)
            pltpu.async_copy(srow, scores_hbm.at[base + r], sem_ss[slot])
        return carry

    lax.fori_loop(0, RPW // CHUNK, chunk_body, 0)
    for i in range(2):
        pltpu.make_async_copy(
            srows[i], scores_hbm.at[base + RPW - 2 + i], sem_ss[i]).wait()


@functools.cache
def _sc_scores_kernel():
    mesh = plsc.VectorSubcoreMesh(
        core_axis_name="c", subcore_axis_name="s",
        num_cores=SC_CORES, num_subcores=SC_SUBCORES)
    return pl.kernel(
        _sc_scores_body,
        out_type=jax.ShapeDtypeStruct((B, D), jnp.float32),
        mesh=mesh,
        scratch_types=[
            pltpu.VMEM((RPW, IDX_PAD), jnp.int32),
            pltpu.VMEM((RPW,), jnp.int32),
            pltpu.VMEM((CHUNK, D), jnp.float32),
            pltpu.VMEM((K, D), jnp.float32),
            pltpu.VMEM((K, D), jnp.float32),
            pltpu.VMEM((D,), jnp.float32),
            pltpu.VMEM((D,), jnp.float32),
            pltpu.SemaphoreType.DMA,
            pltpu.SemaphoreType.DMA,
            pltpu.SemaphoreType.DMA,
            pltpu.SemaphoreType.DMA,
            pltpu.SemaphoreType.DMA,
            pltpu.SemaphoreType.DMA,
        ],
    )


def _select_body(s_ref, x_ref, o_ref):
    bits = lax.bitcast_convert_type(s_ref[...], jnp.int32)
    rb = bits.shape[0]

    def round_body(t, prefix):
        cand = prefix | (1 << (30 - t))
        cnt = jnp.sum((bits >= cand).astype(jnp.int32), axis=1, keepdims=True)
        return jnp.where(cnt >= NUM_DROP, cand, prefix)

    prefix = lax.fori_loop(0, 31, round_body, jnp.zeros((rb, 1), jnp.int32))
    o_ref[...] = jnp.where(bits >= prefix, 0.0, x_ref[...])


def _select_mask(scores, x):
    rb = 256
    return pl.pallas_call(
        _select_body,
        grid=(B // rb,),
        in_specs=[
            pl.BlockSpec((rb, D), lambda i: (i, 0)),
            pl.BlockSpec((rb, D), lambda i: (i, 0)),
        ],
        out_specs=pl.BlockSpec((rb, D), lambda i: (i, 0)),
        out_shape=jax.ShapeDtypeStruct((B, D), jnp.float32),
    )(scores, x)


def kernel(x, y, weight_matrix, prev_output):
    y1 = y.astype(jnp.int32)
    p_pad = jnp.pad(prev_output, ((0, 0), (0, CPAD - C)),
                    constant_values=-jnp.inf)
    idx = _topk_indices(p_pad, y1.reshape(B, 1))
    scores = _sc_scores_kernel()(idx, y1, weight_matrix)
    return _select_mask(scores, x)
